# Initial kernel scaffold; baseline (speedup 1.0000x reference)
import jax
import jax.numpy as jnp
from jax.experimental import pallas as pl


def kernel(x, edge_index, W_self, W_nbr, b, w_out):
    raise NotImplementedError("write your pallas kernel here")


# speedup vs baseline: 5.5669x; 5.5669x over previous
import functools

import jax
import jax.numpy as jnp
from jax import lax
from jax.experimental import pallas as pl
from jax.experimental.pallas import tpu as pltpu
from jax.experimental.pallas import tpu_sc as plsc

N_NODES = 10000
N_EDGES = 320000
D = 128

NC, NS = 2, 16
NW = NC * NS
E_PER_W = N_EDGES // NW
CHUNK = 80
N_CHUNKS = E_PER_W // CHUNK
ROW_CHUNK = 80
N_RCH = N_NODES // ROW_CHUNK


def _sc_aggregate_body(src_hbm, dst_hbm, x_hbm, out_hbm,
                       src_v, dst_v, rows_v, zbuf, acc, sem):
    c = lax.axis_index("c")
    s = lax.axis_index("s")

    zero = jnp.zeros((16,), jnp.float32)

    def zbuf_body(i, carry):
        r = i // 8
        k = (i % 8) * 16
        zbuf[r, pl.ds(k, 16)] = zero
        return carry

    lax.fori_loop(0, ROW_CHUNK * (D // 16), zbuf_body, 0)

    n_mine = (N_RCH - s + NS - 1) // NS

    def zacc_body(t, carry):
        j = s + t * NS
        pltpu.sync_copy(zbuf, acc.at[pl.ds(j * ROW_CHUNK, ROW_CHUNK)])
        return carry

    lax.fori_loop(0, n_mine, zacc_body, 0)
    plsc.subcore_barrier()

    ebase = (c * NS + s) * E_PER_W

    def chunk_body(j, carry):
        off = ebase + j * CHUNK
        pltpu.sync_copy(src_hbm.at[pl.ds(off, CHUNK)], src_v)
        pltpu.sync_copy(dst_hbm.at[pl.ds(off, CHUNK)], dst_v)
        pltpu.async_copy(x_hbm.at[src_v], rows_v, sem).wait()
        pltpu.sync_copy(rows_v, acc.at[dst_v], add=True)
        return carry

    lax.fori_loop(0, N_CHUNKS, chunk_body, 0)
    plsc.subcore_barrier()

    def wb_body(t, carry):
        j = s + t * NS
        pltpu.sync_copy(acc.at[pl.ds(j * ROW_CHUNK, ROW_CHUNK)],
                        out_hbm.at[c, pl.ds(j * ROW_CHUNK, ROW_CHUNK)])
        return carry

    lax.fori_loop(0, n_mine, wb_body, 0)


def _sc_aggregate(edge_index, x):
    mesh = plsc.VectorSubcoreMesh(core_axis_name="c", subcore_axis_name="s")
    k = pl.kernel(
        _sc_aggregate_body,
        out_type=jax.ShapeDtypeStruct((NC, N_NODES, D), jnp.float32),
        mesh=mesh,
        scratch_types=[
            pltpu.VMEM((CHUNK,), jnp.int32),
            pltpu.VMEM((CHUNK,), jnp.int32),
            pltpu.VMEM((CHUNK, D), jnp.float32),
            pltpu.VMEM((ROW_CHUNK, D), jnp.float32),
            pltpu.VMEM_SHARED((N_NODES, D), jnp.float32),
            pltpu.SemaphoreType.DMA,
        ],
    )
    return k(edge_index[0], edge_index[1], x)


def _tc_head_body(x_ref, p_ref, ws_ref, wn_ref, b_ref, wo_ref, out_ref):
    agg = p_ref[0] + p_ref[1]
    h = (jnp.dot(x_ref[...], ws_ref[...], preferred_element_type=jnp.float32)
         + jnp.dot(agg, wn_ref[...], preferred_element_type=jnp.float32)
         + b_ref[...][None, :])
    h = jnp.maximum(h, 0.0)
    out_ref[...] = jnp.sum(h * wo_ref[...][None, :], axis=1)


def _tc_head(x, partials, W_self, W_nbr, b, w_out):
    return pl.pallas_call(
        _tc_head_body,
        out_shape=jax.ShapeDtypeStruct((N_NODES,), jnp.float32),
    )(x, partials, W_self, W_nbr, b, w_out)


def kernel(x, edge_index, W_self, W_nbr, b, w_out):
    partials = _sc_aggregate(edge_index, x)
    return _tc_head(x, partials, W_self, W_nbr, b, w_out)
